# Initial kernel scaffold; baseline (speedup 1.0000x reference)
import jax
import jax.numpy as jnp
from jax.experimental import pallas as pl


def kernel(x, edge_index, edge_type, W1, Q1, K1, b1, WS, QS, KS, bS, W2, Q2, K2, b2, WT, QT, KT, bT, Wr, a_src, a_dst, br):
    raise NotImplementedError("write your pallas kernel here")


# speedup vs baseline: 48.6210x; 48.6210x over previous
import functools

import jax
import jax.numpy as jnp
from jax import lax
from jax.experimental import pallas as pl
from jax.experimental.pallas import tpu as pltpu
from jax.experimental.pallas import tpu_sc as plsc

N = 10000
E = 320000
IN = 128
HEADS = 2
R = 4
HID = 128
OUTS = 33
CPS = 128
NP = 10240
TN = 256
NT = NP // TN

NC = 2
NS = 16
NW = NC * NS
ET = E // NW
CH = 80
NCH = ET // CH
NU = 10240
URT = NU // NS
DCH = NU // NS


@functools.lru_cache(maxsize=None)
def _make_pre(h_, r_, cp):
    def body(x_ref, w_ref, q_ref, k_ref, xr_ref, qt_ref, kt_ref, qm_ref, km_ref):
        xr = jnp.dot(x_ref[...], w_ref[0, 0], preferred_element_type=jnp.float32)
        xr_ref[0, 0] = xr
        qv = jnp.sum(xr * q_ref[0, 0, 0][None, :], axis=1)
        kv = jnp.sum(xr * k_ref[0, 0, 0][None, :], axis=1)
        qt_ref[0, 0] = qv
        kt_ref[0, 0] = kv
        qm_ref[0, 0] = jnp.full((TN,), jnp.max(qv), jnp.float32)
        km_ref[0, 0] = jnp.full((TN,), jnp.max(kv), jnp.float32)

    g = h_ * r_ * NT
    return pl.pallas_call(
        body,
        grid=(h_, r_, NT),
        in_specs=[
            pl.BlockSpec((TN, IN), lambda h, r, i: (i, 0)),
            pl.BlockSpec((1, 1, IN, cp), lambda h, r, i: (h, r, 0, 0)),
            pl.BlockSpec((1, 1, 1, cp), lambda h, r, i: (h, r, 0, 0)),
            pl.BlockSpec((1, 1, 1, cp), lambda h, r, i: (h, r, 0, 0)),
        ],
        out_specs=[
            pl.BlockSpec((1, 1, TN, cp), lambda h, r, i: (h, r, i, 0)),
            pl.BlockSpec((1, 1, TN), lambda h, r, i: ((h * r_ + r) * NT + i, 0, 0)),
            pl.BlockSpec((1, 1, TN), lambda h, r, i: ((h * r_ + r) * NT + i, 0, 0)),
            pl.BlockSpec((1, 1, TN), lambda h, r, i: ((h * r_ + r) * NT + i, 0, 0)),
            pl.BlockSpec((1, 1, TN), lambda h, r, i: ((h * r_ + r) * NT + i, 0, 0)),
        ],
        out_shape=[
            jax.ShapeDtypeStruct((h_, r_, NP, cp), jnp.float32),
            jax.ShapeDtypeStruct((g, 1, TN), jnp.float32),
            jax.ShapeDtypeStruct((g, 1, TN), jnp.float32),
            jax.ShapeDtypeStruct((g, 1, TN), jnp.float32),
            jax.ShapeDtypeStruct((g, 1, TN), jnp.float32),
        ],
    )


@functools.lru_cache(maxsize=None)
def _make_edge(tbl, cp, tstride):
    mesh = plsc.VectorSubcoreMesh(core_axis_name="c", subcore_axis_name="s")
    stg = tbl // NS
    zr = 128
    nvec = cp // 16

    def body(qtab_h, ktab_h, src_h, dst_h, typ_h, xr_h, m_h,
             u0, u1, d0, d1,
             src_v, dst_v, typ_v, gsrc_v, gdst_v, q_v, k_v, ex_v, rows_v,
             m_v, zb_v, zd_v, u_s, den_s, qtab_s, ktab_s, sem0):
        cid = lax.axis_index("c")
        sid = lax.axis_index("s")
        wid = cid * NS + sid

        pltpu.sync_copy(qtab_h.at[pl.ds(sid * stg, stg)], qtab_s.at[pl.ds(sid * stg, stg)])
        pltpu.sync_copy(ktab_h.at[pl.ds(sid * stg, stg)], ktab_s.at[pl.ds(sid * stg, stg)])
        pltpu.sync_copy(m_h, m_v)

        def zrow(i, c):
            for j in range(nvec):
                zb_v[i, pl.ds(j * 16, 16)] = jnp.zeros((16,), jnp.float32)
            return c
        lax.fori_loop(0, zr, zrow, 0)

        def zden(i, c):
            zd_v[pl.ds(i * 16, 16)] = jnp.zeros((16,), jnp.float32)
            return c
        lax.fori_loop(0, DCH // 16, zden, 0)

        for j in range(URT // zr):
            pltpu.sync_copy(zb_v, u_s.at[pl.ds(sid * URT + j * zr, zr)])

        pltpu.sync_copy(zd_v, den_s.at[pl.ds(sid * DCH, DCH)])

        plsc.subcore_barrier()

        mv = m_v[...]

        def chunk(c, carry):
            base = wid * ET + c * CH
            pltpu.sync_copy(src_h.at[pl.ds(base, CH)], src_v)
            pltpu.sync_copy(dst_h.at[pl.ds(base, CH)], dst_v)
            pltpu.sync_copy(typ_h.at[pl.ds(base, CH)], typ_v)
            for i in range(CH // 16):
                sl = pl.ds(i * 16, 16)
                tv = typ_v[sl]
                gsrc_v[sl] = tv * tstride + src_v[sl]
                gdst_v[sl] = tv * tstride + dst_v[sl]
            rcp = pltpu.async_copy(xr_h.at[gsrc_v], rows_v, sem0)
            pltpu.sync_copy(qtab_s.at[gdst_v], q_v)
            pltpu.sync_copy(ktab_s.at[gsrc_v], k_v)
            for i in range(CH // 16):
                sl = pl.ds(i * 16, 16)
                z = q_v[sl] + k_v[sl]
                z = jnp.maximum(z, 0.2 * z)
                ex_v[sl] = jnp.exp(z - mv)
            pltpu.sync_copy(ex_v, den_s.at[dst_v], add=True)
            rcp.wait()

            def scale(g, c2):
                exg = ex_v[pl.ds(g * 16, 16)]
                for i in range(16):
                    s = exg[i]
                    for j in range(nvec):
                        slj = pl.ds(j * 16, 16)
                        rows_v[g * 16 + i, slj] = rows_v[g * 16 + i, slj] * s
                return c2
            lax.fori_loop(0, CH // 16, scale, 0)
            pltpu.sync_copy(rows_v, u_s.at[dst_v], add=True)
            return carry

        lax.fori_loop(0, NCH, chunk, 0)
        plsc.subcore_barrier()

        @pl.when(cid == 0)
        def _():
            pltpu.sync_copy(u_s.at[pl.ds(sid * URT, URT)], u0.at[pl.ds(sid * URT, URT)])

        @pl.when(cid == 1)
        def _():
            pltpu.sync_copy(u_s.at[pl.ds(sid * URT, URT)], u1.at[pl.ds(sid * URT, URT)])

        @pl.when(cid == 0)
        def _():
            pltpu.sync_copy(den_s.at[pl.ds(sid * DCH, DCH)], d0.at[pl.ds(sid * DCH, DCH)])

        @pl.when(cid == 1)
        def _():
            pltpu.sync_copy(den_s.at[pl.ds(sid * DCH, DCH)], d1.at[pl.ds(sid * DCH, DCH)])

    return pl.kernel(
        body,
        out_type=(
            jax.ShapeDtypeStruct((NU, cp), jnp.float32),
            jax.ShapeDtypeStruct((NU, cp), jnp.float32),
            jax.ShapeDtypeStruct((NU,), jnp.float32),
            jax.ShapeDtypeStruct((NU,), jnp.float32),
        ),
        mesh=mesh,
        scratch_types=[
            pltpu.VMEM((CH,), jnp.int32),
            pltpu.VMEM((CH,), jnp.int32),
            pltpu.VMEM((CH,), jnp.int32),
            pltpu.VMEM((CH,), jnp.int32),
            pltpu.VMEM((CH,), jnp.int32),
            pltpu.VMEM((CH,), jnp.float32),
            pltpu.VMEM((CH,), jnp.float32),
            pltpu.VMEM((CH,), jnp.float32),
            pltpu.VMEM((CH, cp), jnp.float32),
            pltpu.VMEM((16,), jnp.float32),
            pltpu.VMEM((zr, cp), jnp.float32),
            pltpu.VMEM((DCH,), jnp.float32),
            pltpu.VMEM_SHARED((NU, cp), jnp.float32),
            pltpu.VMEM_SHARED((NU,), jnp.float32),
            pltpu.VMEM_SHARED((tbl,), jnp.float32),
            pltpu.VMEM_SHARED((tbl,), jnp.float32),
            pltpu.SemaphoreType.DMA,
        ],
    )


RB = 400
NB = N // RB


@functools.lru_cache(maxsize=None)
def _make_post_rgat(cp):
    def body(u00, u01, u10, u11, d00, d01, d10, d11, b_ref, o_ref):
        den0 = d00[0, 0] + d01[0, 0] + 1e-16
        den1 = d10[0, 0] + d11[0, 0] + 1e-16
        num0 = u00[...] + u01[...]
        num1 = u10[...] + u11[...]
        o = 0.5 * (num0 / den0[:, None] + num1 / den1[:, None]) + b_ref[0][None, :]
        o_ref[...] = jnp.where(o > 0, o, jnp.exp(o) - 1.0)

    ub = pl.BlockSpec((RB, cp), lambda i: (i, 0))
    db = pl.BlockSpec((1, 1, RB), lambda i: (i, 0, 0))
    return pl.pallas_call(
        body,
        grid=(NB,),
        in_specs=[ub, ub, ub, ub, db, db, db, db,
                  pl.BlockSpec((1, cp), lambda i: (0, 0))],
        out_specs=ub,
        out_shape=jax.ShapeDtypeStruct((N, cp), jnp.float32),
    )


@functools.lru_cache(maxsize=None)
def _make_post_self(cp):
    def body(u0, u1, d0, d1, b_ref, o_ref):
        den = d0[0, 0] + d1[0, 0] + 1e-16
        num = u0[...] + u1[...]
        o = num / den[:, None] + b_ref[0][None, :]
        o_ref[...] = jnp.where(o > 0, o, jnp.exp(o) - 1.0)

    ub = pl.BlockSpec((RB, cp), lambda i: (i, 0))
    db = pl.BlockSpec((1, 1, RB), lambda i: (i, 0, 0))
    return pl.pallas_call(
        body,
        grid=(NB,),
        in_specs=[ub, ub, db, db, pl.BlockSpec((1, cp), lambda i: (0, 0))],
        out_specs=ub,
        out_shape=jax.ShapeDtypeStruct((N, cp), jnp.float32),
    )


def _leaky(z):
    return jnp.maximum(z, 0.2 * z)


def _pad_rows(a):
    return jnp.pad(a, ((0, NP - a.shape[0]), (0, 0)))


def _prep_w(w, q, k, heads, outc, cp):
    d = w.shape[1]
    wp = w.reshape(R, d, heads, outc).transpose(2, 0, 1, 3)
    qp = q.reshape(R, heads, outc).transpose(1, 0, 2)[:, :, None, :]
    kp = k.reshape(R, heads, outc).transpose(1, 0, 2)[:, :, None, :]
    if cp != outc:
        pad = ((0, 0), (0, 0), (0, 0), (0, cp - outc))
        wp = jnp.pad(wp, pad)
        qp = jnp.pad(qp, pad)
        kp = jnp.pad(kp, pad)
    return wp, qp, kp


def _rgat(xp, w, q, k, b, src, dst, typ, outc, cp):
    wp, qp, kp = _prep_w(w, q, k, HEADS, outc, cp)
    xr, qt, kt, qm, km = _make_pre(HEADS, R, cp)(xp, wp, qp, kp)
    xrh = xr.reshape(HEADS, R * NP, cp)
    qtab = qt.reshape(HEADS, R * NP)
    ktab = kt.reshape(HEADS, R * NP)
    mh = _leaky(qm.reshape(HEADS, -1).max(axis=1) + km.reshape(HEADS, -1).max(axis=1))
    edge = _make_edge(R * NP, cp, NP)
    us, ds_ = [], []
    for h in range(HEADS):
        m16 = jnp.broadcast_to(jnp.reshape(mh[h], (1,)), (16,))
        u0, u1, d0, d1 = edge(qtab[h], ktab[h], src, dst, typ, xrh[h], m16)
        us += [u0, u1]
        ds_ += [d0[:N].reshape(NB, 1, RB), d1[:N].reshape(NB, 1, RB)]
    bp = jnp.pad(b, (0, cp - b.shape[0]))[None, :]
    return _make_post_rgat(cp)(us[0], us[1], us[2], us[3],
                               ds_[0], ds_[1], ds_[2], ds_[3], bp)


def kernel(x, edge_index, edge_type, W1, Q1, K1, b1, WS, QS, KS, bS,
           W2, Q2, K2, b2, WT, QT, KT, bT, Wr, a_src, a_dst, br):
    src = edge_index[0]
    dst = edge_index[1]
    typ = edge_type
    xp = _pad_rows(x)

    x_s = _rgat(xp, W1, Q1, K1, b1, src, dst, typ, HID, HID)
    x_in = _rgat(_pad_rows(x_s), WS, QS, KS, bS, src, dst, typ, OUTS, CPS)[:, :OUTS]

    x_t = _rgat(xp, W2, Q2, K2, b2, src, dst, typ, HID, HID)
    x_out = _rgat(_pad_rows(x_t), WT, QT, KT, bT, src, dst, typ, OUTS, CPS)[:, :OUTS]

    wr = Wr[None, None]
    qp = a_dst[None, None, None, :]
    kp = a_src[None, None, None, :]
    xr, qt, kt, qm, km = _make_pre(1, 1, HID)(xp, wr, qp, kp)
    m0 = _leaky(qm.max() + km.max())
    m16 = jnp.broadcast_to(jnp.reshape(m0, (1,)), (16,))
    u0, u1, d0, d1 = _make_edge(NP, HID, 0)(
        qt.reshape(NP), kt.reshape(NP), src, dst, typ, xr.reshape(NP, HID), m16)
    x_self = _make_post_self(HID)(u0, u1, d0[:N].reshape(NB, 1, RB), d1[:N].reshape(NB, 1, RB),
                                  br[None, :])
    return (x_in, x_out, x_self)
